# Initial kernel scaffold; baseline (speedup 1.0000x reference)
import jax
import jax.numpy as jnp
from jax.experimental import pallas as pl


def kernel(x, emb, W1, b1, W2, b2):
    raise NotImplementedError("write your pallas kernel here")


# speedup vs baseline: 1.2924x; 1.2924x over previous
import functools

import jax
import jax.numpy as jnp
from jax import lax
from jax.experimental import pallas as pl
from jax.experimental.pallas import tpu as pltpu
from jax.experimental.pallas import tpu_sc as plsc

NUM_VOCAB = 100000
NUM_CLASS = 100
EMBED_DIM = 256
HIDDEN_DIM = 128
B, L = 4096, 200

NC, NS, LANES = 2, 16, 16
NW = NC * NS
ROWS_PER_W = B // NW
CHUNK = 104
NCHUNK = 2
LPAD = CHUNK * NCHUNK
NDREG = EMBED_DIM // LANES


def _make_sc_pool():
    mesh = plsc.VectorSubcoreMesh(core_axis_name="c", subcore_axis_name="s")

    @functools.partial(
        pl.kernel,
        mesh=mesh,
        out_type=jax.ShapeDtypeStruct((NW, ROWS_PER_W, EMBED_DIM), jnp.float32),
        scratch_types=[
            pltpu.VMEM((ROWS_PER_W * NCHUNK, CHUNK), jnp.int32),
            pltpu.VMEM((CHUNK, EMBED_DIM), jnp.float32),
            pltpu.VMEM((CHUNK, EMBED_DIM), jnp.float32),
            pltpu.VMEM((ROWS_PER_W, EMBED_DIM), jnp.float32),
            pltpu.SemaphoreType.DMA,
            pltpu.SemaphoreType.DMA,
        ],
    )
    def sc_pool(x_hbm, emb_hbm, h0_hbm, idx_v, buf0, buf1, out_v, sem0, sem1):
        wid = lax.axis_index("s") * NC + lax.axis_index("c")
        pltpu.sync_copy(x_hbm.at[wid], idx_v)
        pltpu.async_copy(emb_hbm.at[idx_v.at[0]], buf0, sem0)
        pltpu.async_copy(emb_hbm.at[idx_v.at[1]], buf1, sem1)

        zero = jnp.zeros((LANES,), jnp.float32)

        def accum(buf, carry):
            def tok_body(t, c):
                ss, cc = c
                ss, cc = list(ss), list(cc)
                for d in range(NDREG):
                    v = buf[t, pl.ds(d * LANES, LANES)]
                    ss[d] = ss[d] + v
                    cc[d] = cc[d] + jnp.where(v != 0.0, 1.0, 0.0)
                return tuple(ss), tuple(cc)

            return lax.fori_loop(0, CHUNK, tok_body, carry)

        def row_body(b, _):
            pltpu.make_async_copy(emb_hbm.at[idx_v.at[0]], buf0, sem0).wait()
            carry = accum(buf0, (tuple(zero for _ in range(NDREG)),) * 2)

            @pl.when(b + 1 < ROWS_PER_W)
            def _():
                pltpu.async_copy(emb_hbm.at[idx_v.at[NCHUNK * (b + 1)]], buf0, sem0)

            pltpu.make_async_copy(emb_hbm.at[idx_v.at[1]], buf1, sem1).wait()
            ss, cc = accum(buf1, carry)

            @pl.when(b + 1 < ROWS_PER_W)
            def _():
                pltpu.async_copy(emb_hbm.at[idx_v.at[NCHUNK * (b + 1) + 1]], buf1, sem1)

            for d in range(NDREG):
                denom = jnp.maximum(cc[d], 1.0)
                out_v[b, pl.ds(d * LANES, LANES)] = ss[d] / denom
            return 0

        lax.fori_loop(0, ROWS_PER_W, row_body, 0)
        pltpu.sync_copy(out_v, h0_hbm.at[wid])

    return sc_pool


_sc_pool = _make_sc_pool()


def _mlp_body(h0_ref, w1_ref, b1_ref, w2_ref, b2_ref, out_ref):
    h1 = jnp.dot(h0_ref[...], w1_ref[...], preferred_element_type=jnp.float32)
    h1 = jnp.maximum(h1 + b1_ref[...], 0.0)
    h2 = jnp.dot(h1, w2_ref[...], preferred_element_type=jnp.float32)
    out_ref[...] = jnp.maximum(h2 + b2_ref[...], 0.0)


def _mlp(h0, W1, b1, W2p, b2p):
    return pl.pallas_call(
        _mlp_body,
        out_shape=jax.ShapeDtypeStruct((B, 128), jnp.float32),
    )(h0, W1, b1.reshape(1, HIDDEN_DIM), W2p, b2p.reshape(1, 128))


def kernel(x, emb, W1, b1, W2, b2):
    x3 = jnp.pad(x.astype(jnp.int32), ((0, 0), (0, LPAD - L))).reshape(
        NW, ROWS_PER_W * NCHUNK, CHUNK)
    h0 = _sc_pool(x3, emb).reshape(B, EMBED_DIM)
    W2p = jnp.pad(W2, ((0, 0), (0, 128 - NUM_CLASS)))
    b2p = jnp.pad(b2, (0, 128 - NUM_CLASS))
    out = _mlp(h0, W1, b1, W2p, b2p)
    return out[:, :NUM_CLASS]


# speedup vs baseline: 4.7860x; 3.7032x over previous
import functools

import jax
import jax.numpy as jnp
from jax import lax
from jax.experimental import pallas as pl
from jax.experimental.pallas import tpu as pltpu
from jax.experimental.pallas import tpu_sc as plsc

NUM_VOCAB = 100000
NUM_CLASS = 100
EMBED_DIM = 256
HIDDEN_DIM = 128
B, L = 4096, 200

NC, NS, LANES = 2, 16, 16
NW = NC * NS
ROWS_PER_W = B // NW
CH0, CH1 = 96, 104
NDREG = EMBED_DIM // LANES


def _make_sc_pool():
    mesh = plsc.VectorSubcoreMesh(core_axis_name="c", subcore_axis_name="s")

    @functools.partial(
        pl.kernel,
        mesh=mesh,
        out_type=jax.ShapeDtypeStruct((NW, ROWS_PER_W, EMBED_DIM), jnp.float32),
        scratch_types=[
            pltpu.VMEM((ROWS_PER_W * L,), jnp.int32),
            pltpu.VMEM((CH0, EMBED_DIM), jnp.float32),
            pltpu.VMEM((CH1, EMBED_DIM), jnp.float32),
            pltpu.VMEM((ROWS_PER_W, EMBED_DIM), jnp.float32),
            pltpu.SemaphoreType.DMA,
            pltpu.SemaphoreType.DMA,
        ],
    )
    def sc_pool(x_hbm, emb_hbm, h0_hbm, idx_v, buf0, buf1, out_v, sem0, sem1):
        wid = lax.axis_index("s") * NC + lax.axis_index("c")
        pltpu.sync_copy(x_hbm.at[wid], idx_v)
        pltpu.async_copy(emb_hbm.at[idx_v.at[pl.ds(0, CH0)]], buf0, sem0)
        pltpu.async_copy(emb_hbm.at[idx_v.at[pl.ds(CH0, CH1)]], buf1, sem1)

        zero = jnp.zeros((LANES,), jnp.float32)

        def accum(buf, chunk, carry):
            def tok_body(t, c):
                ss, zc = c
                ss, zc = list(ss), list(zc)
                for d in range(NDREG):
                    v = buf[t, pl.ds(d * LANES, LANES)]
                    ss[d] = ss[d] + v
                    zc[d] = zc[d] + jnp.where(v == 0.0, 1.0, 0.0)
                return tuple(ss), tuple(zc)

            return lax.fori_loop(0, chunk, tok_body, carry)

        def row_body(b, _):
            pltpu.make_async_copy(emb_hbm.at[idx_v.at[pl.ds(0, CH0)]], buf0, sem0).wait()
            carry = accum(buf0, CH0, (tuple(zero for _ in range(NDREG)),) * 2)

            @pl.when(b + 1 < ROWS_PER_W)
            def _():
                pltpu.async_copy(emb_hbm.at[idx_v.at[pl.ds((b + 1) * L, CH0)]], buf0, sem0)

            pltpu.make_async_copy(emb_hbm.at[idx_v.at[pl.ds(CH0, CH1)]], buf1, sem1).wait()
            ss, zcs = accum(buf1, CH1, carry)

            @pl.when(b + 1 < ROWS_PER_W)
            def _():
                pltpu.async_copy(emb_hbm.at[idx_v.at[pl.ds((b + 1) * L + CH0, CH1)]], buf1, sem1)

            for d in range(NDREG):
                cnt = float(L) - zcs[d]
                out_v[b, pl.ds(d * LANES, LANES)] = ss[d] / jnp.maximum(cnt, 1.0)
            return 0

        lax.fori_loop(0, ROWS_PER_W, row_body, 0)
        pltpu.sync_copy(out_v, h0_hbm.at[wid])

    return sc_pool


_sc_pool = _make_sc_pool()


def _mlp_body(h0_ref, w1_ref, b1_ref, w2_ref, b2_ref, out_ref):
    h1 = jnp.dot(h0_ref[...], w1_ref[...], preferred_element_type=jnp.float32)
    h1 = jnp.maximum(h1 + b1_ref[...], 0.0)
    h2 = jnp.dot(h1, w2_ref[...], preferred_element_type=jnp.float32)
    out_ref[...] = jnp.maximum(h2 + b2_ref[...], 0.0)


def _mlp(h0, W1, b1, W2p, b2p):
    return pl.pallas_call(
        _mlp_body,
        out_shape=jax.ShapeDtypeStruct((B, 128), jnp.float32),
    )(h0, W1, b1.reshape(1, HIDDEN_DIM), W2p, b2p.reshape(1, 128))


def kernel(x, emb, W1, b1, W2, b2):
    x3 = x.astype(jnp.int32).reshape(NW, ROWS_PER_W * L)
    h0 = _sc_pool(x3, emb).reshape(B, EMBED_DIM)
    W2p = jnp.pad(W2, ((0, 0), (0, 128 - NUM_CLASS)))
    b2p = jnp.pad(b2, (0, 128 - NUM_CLASS))
    out = _mlp(h0, W1, b1, W2p, b2p)
    return out[:, :NUM_CLASS]


# speedup vs baseline: 5.3944x; 1.1271x over previous
import functools

import jax
import jax.numpy as jnp
from jax import lax
from jax.experimental import pallas as pl
from jax.experimental.pallas import tpu as pltpu
from jax.experimental.pallas import tpu_sc as plsc

NUM_VOCAB = 100000
NUM_CLASS = 100
EMBED_DIM = 256
HIDDEN_DIM = 128
B, L = 4096, 200

NC, NS, LANES = 2, 16, 16
NW = NC * NS
ROWS_PER_W = B // NW
NCHUNK = 5
CHUNK = L // NCHUNK
NDREG = EMBED_DIM // LANES


def _make_sc_pool():
    mesh = plsc.VectorSubcoreMesh(core_axis_name="c", subcore_axis_name="s")

    @functools.partial(
        pl.kernel,
        mesh=mesh,
        out_type=jax.ShapeDtypeStruct((NW, ROWS_PER_W, EMBED_DIM), jnp.float32),
        scratch_types=(
            [pltpu.VMEM((ROWS_PER_W * L,), jnp.int32)]
            + [pltpu.VMEM((CHUNK, EMBED_DIM), jnp.float32)
               for _ in range(NCHUNK)]
            + [pltpu.VMEM((ROWS_PER_W, EMBED_DIM), jnp.float32)]
            + [pltpu.SemaphoreType.DMA for _ in range(NCHUNK)]
        ),
    )
    def sc_pool(x_hbm, emb_hbm, h0_hbm, idx_v, *rest):
        bufs = rest[:NCHUNK]
        out_v = rest[NCHUNK]
        sems = rest[NCHUNK + 1:]
        wid = lax.axis_index("s") * NC + lax.axis_index("c")
        pltpu.sync_copy(x_hbm.at[wid], idx_v)
        for j in range(NCHUNK):
            pltpu.async_copy(emb_hbm.at[idx_v.at[pl.ds(j * CHUNK, CHUNK)]],
                             bufs[j], sems[j])

        zero = jnp.zeros((LANES,), jnp.float32)

        def accum(buf, carry):
            def tok_body(t, c):
                ss, zc = c
                ss, zc = list(ss), list(zc)
                for d in range(NDREG):
                    v = buf[t, pl.ds(d * LANES, LANES)]
                    ss[d] = ss[d] + v
                    zc[d] = zc[d] + jnp.where(v == 0.0, 1.0, 0.0)
                return tuple(ss), tuple(zc)

            return lax.fori_loop(0, CHUNK, tok_body, carry)

        def row_body(b, _):
            carry = (tuple(zero for _ in range(NDREG)),) * 2
            for j in range(NCHUNK):
                pltpu.make_async_copy(
                    emb_hbm.at[idx_v.at[pl.ds(j * CHUNK, CHUNK)]],
                    bufs[j], sems[j]).wait()
                carry = accum(bufs[j], carry)

                @pl.when(b + 1 < ROWS_PER_W)
                def _(j=j):
                    pltpu.async_copy(
                        emb_hbm.at[idx_v.at[pl.ds((b + 1) * L + j * CHUNK, CHUNK)]],
                        bufs[j], sems[j])

            ss, zcs = carry
            for d in range(NDREG):
                cnt = float(L) - zcs[d]
                out_v[b, pl.ds(d * LANES, LANES)] = ss[d] / jnp.maximum(cnt, 1.0)
            return 0

        lax.fori_loop(0, ROWS_PER_W, row_body, 0)
        pltpu.sync_copy(out_v, h0_hbm.at[wid])

    return sc_pool


_sc_pool = _make_sc_pool()


def _mlp_body(h0_ref, w1_ref, b1_ref, w2_ref, b2_ref, out_ref):
    h1 = jnp.dot(h0_ref[...], w1_ref[...], preferred_element_type=jnp.float32)
    h1 = jnp.maximum(h1 + b1_ref[...], 0.0)
    h2 = jnp.dot(h1, w2_ref[...], preferred_element_type=jnp.float32)
    out_ref[...] = jnp.maximum(h2 + b2_ref[...], 0.0)


def _mlp(h0, W1, b1, W2p, b2p):
    return pl.pallas_call(
        _mlp_body,
        out_shape=jax.ShapeDtypeStruct((B, 128), jnp.float32),
    )(h0, W1, b1.reshape(1, HIDDEN_DIM), W2p, b2p.reshape(1, 128))


def kernel(x, emb, W1, b1, W2, b2):
    x3 = x.astype(jnp.int32).reshape(NW, ROWS_PER_W * L)
    h0 = _sc_pool(x3, emb).reshape(B, EMBED_DIM)
    W2p = jnp.pad(W2, ((0, 0), (0, 128 - NUM_CLASS)))
    b2p = jnp.pad(b2, (0, 128 - NUM_CLASS))
    out = _mlp(h0, W1, b1, W2p, b2p)
    return out[:, :NUM_CLASS]
